# Initial kernel scaffold; baseline (speedup 1.0000x reference)
import functools

import jax
import jax.numpy as jnp
from jax import lax
from jax.experimental import pallas as pl
from jax.experimental.pallas import tpu as pltpu
from jax.experimental.pallas import tpu_sc as plsc

N = 10000
D = 128
DP = 144
NACC = 10016
E = 320000
NC, NS = 2, 16
NW = NC * NS
EPT = 10240
EPAD = EPT * NW
CH = 128
NCH = EPT // CH
NBUF = 4
NGRP = NCH // NBUF
ZROWS = NACC // NS


def _sc_body(xa_hbm, src_hbm, dst_hbm, out_hbm,
             src_v, dst_v, rows, acc, *sems):
    c = lax.axis_index("c")
    s = lax.axis_index("s")
    wid = s * NC + c
    gsems = sems[:NBUF]
    ssems = sems[NBUF:]

    zb = rows.at[0]
    def zrow(i, carry):
        r = i // (DP // 16)
        col = (i % (DP // 16)) * 16
        zb[r, pl.ds(col, 16)] = jnp.zeros((16,), jnp.float32)
        return carry
    lax.fori_loop(0, CH * DP // 16, zrow, 0)
    row0 = s * ZROWS
    nfull = ZROWS // CH
    for j in range(nfull):
        pltpu.sync_copy(zb, acc.at[pl.ds(row0 + j * CH, CH)])
    rem = ZROWS - nfull * CH
    if rem:
        pltpu.sync_copy(zb.at[pl.ds(0, rem)], acc.at[pl.ds(row0 + nfull * CH, rem)])
    plsc.subcore_barrier()

    pltpu.sync_copy(src_hbm.at[wid], src_v)
    pltpu.sync_copy(dst_hbm.at[wid], dst_v)

    def fire_gather(g, b):
        pltpu.async_copy(xa_hbm.at[src_v.at[pl.ds(g * CH, CH)]], rows.at[b], gsems[b])

    def wait_gather(g, b):
        pltpu.make_async_copy(
            xa_hbm.at[src_v.at[pl.ds(g * CH, CH)]], rows.at[b], gsems[b]).wait()

    def fire_scatter(g, b):
        pltpu.async_copy(rows.at[b], acc.at[dst_v.at[g]], ssems[b], add=True)

    def wait_scatter(g, b):
        pltpu.make_async_copy(rows.at[b], acc.at[dst_v.at[g]], ssems[b]).wait()

    for b in range(NBUF):
        fire_gather(b, b)

    def group(gi, carry):
        for b in range(NBUF):
            g = gi * NBUF + b
            wait_gather(g, b)
            fire_scatter(g, b)
            wait_scatter(g, b)
            fire_gather(g + NBUF, b)
        return carry
    lax.fori_loop(0, NGRP - 1, group, 0)

    for b in range(NBUF):
        g = (NGRP - 1) * NBUF + b
        wait_gather(g, b)
        fire_scatter(g, b)
        wait_scatter(g, b)

    plsc.subcore_barrier()

    pltpu.sync_copy(acc.at[pl.ds(row0, ZROWS)], out_hbm.at[c, pl.ds(row0, ZROWS)])


_sc_scatter = functools.partial(
    pl.kernel,
    out_type=jax.ShapeDtypeStruct((NC, NACC, DP), jnp.float32),
    mesh=plsc.VectorSubcoreMesh(
        core_axis_name="c", subcore_axis_name="s", num_cores=NC, num_subcores=NS),
    scratch_types=[
        pltpu.VMEM((EPT,), jnp.int32),
        pltpu.VMEM((NCH, CH), jnp.int32),
        pltpu.VMEM((NBUF, CH, DP), jnp.float32),
        pltpu.VMEM_SHARED((NACC, DP), jnp.float32),
    ] + [pltpu.SemaphoreType.DMA] * (2 * NBUF),
)(_sc_body)


BN = 1000


def _tc_body(x_ref, p0_ref, p1_ref, w1a_ref, w1b_ref, w2a_ref, w2b_ref, o_ref):
    xb = x_ref[...]
    p0 = p0_ref[...]
    p1 = p1_ref[...]
    sv = p0[:, :D] + p1[:, :D]
    dg = p0[:, D:D + 1] + p1[:, D:D + 1]
    agg = (jnp.dot(sv, w1a_ref[...], preferred_element_type=jnp.float32)
           + jnp.dot(xb * dg, w1b_ref[...], preferred_element_type=jnp.float32))
    agg = agg * jnp.float32(1.0 / 9.0)
    o_ref[...] = (
        jnp.dot(jnp.maximum(xb, 0.0), w2a_ref[...], preferred_element_type=jnp.float32)
        + jnp.dot(jnp.maximum(agg, 0.0), w2b_ref[...], preferred_element_type=jnp.float32))


def _tc_finish(x, p0, p1, w1a, w1b, w2a, w2b):
    wspec = pl.BlockSpec((D, D), lambda i: (0, 0))
    return pl.pallas_call(
        _tc_body,
        grid=(N // BN,),
        in_specs=[
            pl.BlockSpec((BN, D), lambda i: (i, 0)),
            pl.BlockSpec((BN, DP), lambda i: (i, 0)),
            pl.BlockSpec((BN, DP), lambda i: (i, 0)),
            wspec, wspec, wspec, wspec,
        ],
        out_specs=pl.BlockSpec((BN, D), lambda i: (i, 0)),
        out_shape=jax.ShapeDtypeStruct((N, D), jnp.float32),
    )(x, p0, p1, w1a, w1b, w2a, w2b)


def kernel(x, edge_index, W1, W2):
    src = edge_index[:, 0].astype(jnp.int32)
    dst = edge_index[:, 1].astype(jnp.int32)
    src_p = jnp.concatenate([src, jnp.zeros((EPAD - E,), jnp.int32)]).reshape(NW, EPT)
    dst_p = jnp.concatenate(
        [dst, jnp.full((EPAD - E,), N, jnp.int32)]).reshape(NW, NCH, CH)
    xa = jnp.concatenate(
        [x, jnp.ones((N, 1), jnp.float32), jnp.zeros((N, DP - D - 1), jnp.float32)],
        axis=1)
    partials = _sc_scatter(xa, src_p, dst_p)
    return _tc_finish(x, partials[0], partials[1],
                      W1[:D], W1[D:], W2[:D], W2[D:])


# speedup vs baseline: 7.4999x; 7.4999x over previous
import functools

import jax
import jax.numpy as jnp
from jax import lax
from jax.experimental import pallas as pl
from jax.experimental.pallas import tpu as pltpu
from jax.experimental.pallas import tpu_sc as plsc

N = 10000
D = 128
WL = 80
NACC = 10112
E = 320000
NC, NS = 2, 16
EPT = 20480
CH = 128
NCH = EPT // CH
NHALF = 2
HCH = NCH // NHALF
NBUF = 4
NGRP = HCH // NBUF
ZROWS = NACC // NS


def _sc_body(xlo_hbm, xhi_hbm, src_hbm, dst_hbm, out_hbm,
             src_v, dst_v, rows, acc, *sems):
    c = lax.axis_index("c")
    s = lax.axis_index("s")
    gsems = sems[:NBUF]
    ssems = sems[NBUF:]

    zb = rows.at[0]
    def zrow(i, carry):
        r = i // (WL // 16)
        col = (i % (WL // 16)) * 16
        zb[r, pl.ds(col, 16)] = jnp.zeros((16,), jnp.float32)
        return carry
    lax.fori_loop(0, CH * WL // 16, zrow, 0)
    row0 = s * ZROWS
    nfull = ZROWS // CH
    for j in range(nfull):
        pltpu.sync_copy(zb, acc.at[pl.ds(row0 + j * CH, CH)])
    rem = ZROWS - nfull * CH
    if rem:
        pltpu.sync_copy(zb.at[pl.ds(0, rem)], acc.at[pl.ds(row0 + nfull * CH, rem)])
    plsc.subcore_barrier()

    def run_edges(table):
        def fire_gather(g, b):
            pltpu.async_copy(
                table.at[src_v.at[pl.ds(g * CH, CH)]], rows.at[b], gsems[b])

        def wait_gather(g, b):
            pltpu.make_async_copy(
                table.at[src_v.at[pl.ds(g * CH, CH)]], rows.at[b], gsems[b]).wait()

        def fire_scatter(g, b):
            pltpu.async_copy(rows.at[b], acc.at[dst_v.at[g]], ssems[b], add=True)

        def wait_scatter(g, b):
            pltpu.make_async_copy(rows.at[b], acc.at[dst_v.at[g]], ssems[b]).wait()

        for h in range(NHALF):
            pltpu.sync_copy(src_hbm.at[s, pl.ds(h * HCH * CH, HCH * CH)], src_v)
            pltpu.sync_copy(dst_hbm.at[s, pl.ds(h * HCH, HCH)], dst_v)

            for b in range(NBUF):
                fire_gather(b, b)

            def group(gi, carry):
                for b in range(NBUF):
                    g = gi * NBUF + b
                    wait_gather(g, b)
                    fire_scatter(g, b)
                    wait_scatter(g, b)
                    fire_gather(g + NBUF, b)
                return carry
            lax.fori_loop(0, NGRP - 1, group, 0)

            for b in range(NBUF):
                g = (NGRP - 1) * NBUF + b
                wait_gather(g, b)
                fire_scatter(g, b)
                wait_scatter(g, b)

    @pl.when(c == 0)
    def _():
        run_edges(xlo_hbm)

    @pl.when(c == 1)
    def _():
        run_edges(xhi_hbm)

    plsc.subcore_barrier()

    pltpu.sync_copy(acc.at[pl.ds(row0, ZROWS)], out_hbm.at[c, pl.ds(row0, ZROWS)])


@functools.cache
def _sc_scatter():
    return pl.kernel(
        _sc_body,
        out_type=jax.ShapeDtypeStruct((NC, NACC, WL), jnp.float32),
        mesh=plsc.VectorSubcoreMesh(
            core_axis_name="c", subcore_axis_name="s",
            num_cores=NC, num_subcores=NS),
        scratch_types=[
            pltpu.VMEM((HCH * CH,), jnp.int32),
            pltpu.VMEM((HCH, CH), jnp.int32),
            pltpu.VMEM((NBUF, CH, WL), jnp.float32),
            pltpu.VMEM_SHARED((NACC, WL), jnp.float32),
        ] + [pltpu.SemaphoreType.DMA] * (2 * NBUF),
        compiler_params=pltpu.CompilerParams(use_tc_tiling_on_sc=False),
    )


BN = 1000


def _tc_body(x_ref, plo_ref, phi_ref, w1al_ref, w1ah_ref, w1b_ref,
             w2a_ref, w2b_ref, o_ref):
    xb = x_ref[...]
    plo = plo_ref[...]
    phi = phi_ref[...]
    dg = phi[:, D - WL:D - WL + 1]
    agg = (jnp.dot(plo, w1al_ref[...], preferred_element_type=jnp.float32)
           + jnp.dot(phi[:, :D - WL], w1ah_ref[...], preferred_element_type=jnp.float32)
           + jnp.dot(xb * dg, w1b_ref[...], preferred_element_type=jnp.float32))
    agg = agg * jnp.float32(1.0 / 9.0)
    o_ref[...] = (
        jnp.dot(jnp.maximum(xb, 0.0), w2a_ref[...], preferred_element_type=jnp.float32)
        + jnp.dot(jnp.maximum(agg, 0.0), w2b_ref[...], preferred_element_type=jnp.float32))


def _tc_finish(x, plo, phi, w1al, w1ah, w1b, w2a, w2b):
    def wspec(k):
        return pl.BlockSpec((k, D), lambda i: (0, 0))
    return pl.pallas_call(
        _tc_body,
        grid=(N // BN,),
        in_specs=[
            pl.BlockSpec((BN, D), lambda i: (i, 0)),
            pl.BlockSpec((BN, WL), lambda i: (i, 0)),
            pl.BlockSpec((BN, WL), lambda i: (i, 0)),
            wspec(WL), wspec(D - WL), wspec(D), wspec(D), wspec(D),
        ],
        out_specs=pl.BlockSpec((BN, D), lambda i: (i, 0)),
        out_shape=jax.ShapeDtypeStruct((N, D), jnp.float32),
    )(x, plo, phi, w1al, w1ah, w1b, w2a, w2b)


def kernel(x, edge_index, W1, W2):
    src = edge_index[:, 0].astype(jnp.int32)
    dst = edge_index[:, 1].astype(jnp.int32)
    pad = EPT - E // NS
    src_p = jnp.concatenate(
        [src.reshape(NS, E // NS), jnp.zeros((NS, pad), jnp.int32)], axis=1)
    dst_p = jnp.concatenate(
        [dst.reshape(NS, E // NS), jnp.full((NS, pad), N, jnp.int32)],
        axis=1).reshape(NS, NCH, CH)
    xa = jnp.concatenate(
        [x, jnp.ones((N, 1), jnp.float32),
         jnp.zeros((N, 2 * WL - D - 1), jnp.float32)], axis=1)
    partials = _sc_scatter()(xa[:, :WL], xa[:, WL:], src_p, dst_p)
    return _tc_finish(x, partials[0], partials[1],
                      W1[:WL], W1[WL:D], W1[D:], W2[:D], W2[D:])
